# baseline (speedup 1.0000x reference)
import jax
import jax.numpy as jnp
from jax.experimental import pallas as pl
from jax.experimental.pallas import tpu as pltpu

B, S, D_MODEL, D_SPACE = 4, 4096, 2048, 64
N_FQK = 512
TM = 512


def _router_kernel(x_ref, w_ref, b_ref, emb_ref, out_ref):
    h = jnp.dot(x_ref[...].astype(jnp.bfloat16),
                w_ref[...].astype(jnp.bfloat16),
                preferred_element_type=jnp.float32)
    h = h + b_ref[...]
    emb = emb_ref[...]
    ss = jnp.sum(emb * emb, axis=1)
    scale = jax.lax.rsqrt(jnp.maximum(ss, 1e-24))
    logits = jax.lax.dot_general(
        h.astype(jnp.bfloat16), emb.astype(jnp.bfloat16),
        (((1,), (1,)), ((), ())),
        preferred_element_type=jnp.float32)
    out_ref[...] = logits * scale[None, :]


def kernel(x, W, b, neuron_emb):
    T = B * S
    x2 = x.reshape(T, D_MODEL)
    emb = neuron_emb[:N_FQK]
    b2 = b.reshape(1, D_SPACE)
    grid = (T // TM,)
    out = pl.pallas_call(
        _router_kernel,
        grid=grid,
        in_specs=[
            pl.BlockSpec((TM, D_MODEL), lambda i: (i, 0)),
            pl.BlockSpec((D_MODEL, D_SPACE), lambda i: (0, 0)),
            pl.BlockSpec((1, D_SPACE), lambda i: (0, 0)),
            pl.BlockSpec((N_FQK, D_SPACE), lambda i: (0, 0)),
        ],
        out_specs=pl.BlockSpec((TM, N_FQK), lambda i: (i, 0)),
        out_shape=jax.ShapeDtypeStruct((T, N_FQK), jnp.float32),
        compiler_params=pltpu.CompilerParams(
            dimension_semantics=("parallel",)),
    )(x2, W, b2, emb)
    return out.reshape(B, S, N_FQK)


# speedup vs baseline: 1.1818x; 1.1818x over previous
import jax
import jax.numpy as jnp
from jax.experimental import pallas as pl
from jax.experimental.pallas import tpu as pltpu

B, S, D_MODEL, D_SPACE = 4, 4096, 2048, 64
N_FQK = 512
TM = 2048


def _router_kernel(x_ref, w_ref, b_ref, emb_ref, out_ref):
    h = jnp.dot(x_ref[...].astype(jnp.bfloat16),
                w_ref[...].astype(jnp.bfloat16),
                preferred_element_type=jnp.float32)
    h = h + b_ref[...]
    emb = emb_ref[...]
    ss = jnp.sum(emb * emb, axis=1)
    scale = jax.lax.rsqrt(jnp.maximum(ss, 1e-24))
    logits = jax.lax.dot_general(
        h.astype(jnp.bfloat16), emb.astype(jnp.bfloat16),
        (((1,), (1,)), ((), ())),
        preferred_element_type=jnp.float32)
    out_ref[...] = logits * scale[None, :]


def kernel(x, W, b, neuron_emb):
    T = B * S
    x2 = x.reshape(T, D_MODEL)
    emb = neuron_emb[:N_FQK]
    b2 = b.reshape(1, D_SPACE)
    grid = (T // TM,)
    out = pl.pallas_call(
        _router_kernel,
        grid=grid,
        in_specs=[
            pl.BlockSpec((TM, D_MODEL), lambda i: (i, 0)),
            pl.BlockSpec((D_MODEL, D_SPACE), lambda i: (0, 0)),
            pl.BlockSpec((1, D_SPACE), lambda i: (0, 0)),
            pl.BlockSpec((N_FQK, D_SPACE), lambda i: (0, 0)),
        ],
        out_specs=pl.BlockSpec((TM, N_FQK), lambda i: (i, 0)),
        out_shape=jax.ShapeDtypeStruct((T, N_FQK), jnp.float32),
        compiler_params=pltpu.CompilerParams(
            dimension_semantics=("parallel",)),
    )(x2, W, b2, emb)
    return out.reshape(B, S, N_FQK)
